# baseline (speedup 1.0000x reference)
import dataclasses
import numpy as np
import jax
import jax.numpy as jnp
from jax import lax
from jax.experimental import pallas as pl
from jax.experimental.pallas import tpu as pltpu
from jax.experimental.pallas import tpu_sc as plsc

B, L, D = 4, 2048, 1024
VOCAB = 100000
N_HEADS = 16
HEAD_DIM = 64
E = N_HEADS * HEAD_DIM
HC = 4
KCONV = 4
PAD_ID = 2
BL = B * L

T = 256
NL = L // T
NW = 32
CHUNK = BL // NW
GW = 128
NG = CHUNK // GW
NCH = N_HEADS * NG

NTC = VOCAB // 128 + 1
TC_PER_W = NTC // 2
VP = VOCAB // 2


def _hash_coeffs() -> np.ndarray:
    rs = np.random.RandomState(0)
    m2 = rs.randint(1, 2 ** 31 - 1, size=(8, 2)).astype(np.uint64)
    m3 = rs.randint(1, 2 ** 31 - 1, size=(8, 3)).astype(np.uint64)
    mod = np.uint64(2 ** 32)
    p = np.uint64(1000003)
    c = np.zeros((16, 3), dtype=np.uint64)
    c[:8, 0] = (m2[:, 0] * p) % mod
    c[:8, 1] = m2[:, 1] % mod
    c[8:, 0] = (m3[:, 0] * p % mod * p) % mod
    c[8:, 1] = (m3[:, 1] * p) % mod
    c[8:, 2] = m3[:, 2] % mod
    return c.astype(np.uint32).view(np.int32).reshape(16, 3)


_C = _hash_coeffs()


def _sc_params() -> pltpu.CompilerParams:
    cp = pltpu.CompilerParams(use_tc_tiling_on_sc=True)
    if "needs_layout_passes" in pltpu.CompilerParams.__dataclass_fields__:
        cp = dataclasses.replace(cp, needs_layout_passes=False)
    return cp


def _sc_transpose_body_v2(tabt_hbm, out_hbm, in0, in1, out0, out1, isem, osem):
    wid = lax.axis_index("c") * 16 + lax.axis_index("s")
    h = lax.shift_right_logical(wid, 1)
    wh = lax.bitwise_and(wid, 1)
    tc0 = wh * TC_PER_W

    iota = jnp.arange(16, dtype=jnp.int32)
    rowvec = [qm * 16 + iota for qm in range(4)]

    def transpose(in_buf, out_buf, nvp):
        @plsc.parallel_loop(0, nvp, unroll=16)
        def _(vp):
            col0 = jnp.zeros((16,), jnp.int32) + 2 * vp
            col1 = col0 + 1
            for q in range(8):
                col = col1 if q >= 4 else col0
                x = plsc.load_gather(in_buf, [rowvec[q % 4], col])
                out_buf[vp, pl.ds(q * 16, 16)] = x

    def fire_in(tc, buf):
        return pltpu.async_copy(
            tabt_hbm.at[h, :, pl.ds(tc * 128, 128)], buf.at[:, pl.ds(0, 128)],
            isem)

    def fire_out(tc, buf, nvp):
        return pltpu.async_copy(
            buf.at[pl.ds(0, nvp)], out_hbm.at[h, pl.ds(tc * 64, nvp)], osem)

    ins = (in0, in1)
    outs = (out0, out1)
    NFULL = TC_PER_W - 1

    din = [fire_in(tc0 + 0, ins[0]), fire_in(tc0 + 1, ins[1])]

    @pl.loop(0, NFULL // 2)
    def _(k2):
        k = k2 * 2
        for sl in range(2):
            tc = tc0 + k + sl
            din[sl].wait()
            transpose(ins[sl], outs[sl], 64)
            fire_out(tc, outs[sl], 64).wait()
            nk = k + sl + 2

            @pl.when(nk < NFULL)
            def _(sl=sl, nk=nk):
                fire_in(tc0 + nk, ins[sl])

    tail_tc = tc0 + NFULL

    d = fire_in(tail_tc, ins[0])
    d.wait()

    @pl.when(wh == 0)
    def _():
        transpose(ins[0], outs[0], 64)
        fire_out(tail_tc, outs[0], 64).wait()

    @pl.when(wh == 1)
    def _():
        transpose(ins[0], outs[0], 16)
        fire_out(tail_tc, outs[0], 16).wait()


def _sc_transpose(tabt):
    mesh = plsc.VectorSubcoreMesh(core_axis_name="c", subcore_axis_name="s")
    f = pl.kernel(
        _sc_transpose_body_v2,
        mesh=mesh,
        out_type=jax.ShapeDtypeStruct((N_HEADS, VP, 128), jnp.float32),
        scratch_types=[
            pltpu.VMEM((HEAD_DIM, 129), jnp.float32),
            pltpu.VMEM((HEAD_DIM, 129), jnp.float32),
            pltpu.VMEM((HEAD_DIM, 128), jnp.float32),
            pltpu.VMEM((HEAD_DIM, 128), jnp.float32),
            pltpu.SemaphoreType.DMA,
            pltpu.SemaphoreType.DMA,
        ],
        compiler_params=_sc_params(),
    )
    return f(tabt)


def _sc_gather_body(ids_hbm, tab_hbm, out_hbm, par_hbm,
                    ids_v, idx_v, par_v, rows0, rows1, gsem, wsem, psem):
    wid = lax.axis_index("c") * 16 + lax.axis_index("s")
    base = wid * CHUNK

    pltpu.sync_copy(ids_hbm.at[pl.ds(base, CHUNK)], ids_v.at[pl.ds(16, CHUNK)])
    at_row_start = lax.rem(wid, L // CHUNK) == 0

    @pl.when(at_row_start)
    def _():
        ids_v[pl.ds(0, 16)] = jnp.full((16,), PAD_ID, jnp.int32)

    @pl.when(jnp.logical_not(at_row_start))
    def _():
        pltpu.sync_copy(ids_hbm.at[pl.ds(base - 16, 16)], ids_v.at[pl.ds(0, 16)])

    for h in range(N_HEADS):
        c0 = jnp.int32(int(_C[h, 0]))
        c1 = jnp.int32(int(_C[h, 1]))
        c2 = jnp.int32(int(_C[h, 2]))
        for hf in range(NG):

            @pl.loop(0, GW // 16)
            def _(i, h=h, hf=hf, c0=c0, c1=c1, c2=c2):
                s = 16 + hf * GW + i * 16
                t0 = ids_v[pl.ds(s, 16)]
                t1 = ids_v[pl.ds(s - 1, 16)]
                acc = t0 * c0 + t1 * c1
                if int(_C[h, 2]) != 0:
                    t2 = ids_v[pl.ds(s - 2, 16)]
                    acc = acc + t2 * c2
                accu = plsc.bitcast(acc, jnp.uint32)
                v = plsc.bitcast(accu % jnp.uint32(VOCAB), jnp.int32)
                j = h * NG + hf
                idx_v[j, pl.ds(i * 16, 16)] = lax.shift_right_logical(v, 1)
                par_v[j, pl.ds(i * 16, 16)] = lax.bitwise_and(v, 1).astype(
                    jnp.float32)

    pd = []
    for j in range(NCH):
        h, hf = j // NG, j % NG
        pd.append(pltpu.async_copy(
            par_v.at[j], par_hbm.at[h, pl.ds(base + hf * GW, GW)], psem))

    rows = (rows0, rows1)
    gd = [None] * NCH
    wd = [None] * NCH

    def _write(j):
        h, hf = j // NG, j % NG
        return pltpu.async_copy(
            rows[j % 2],
            out_hbm.at[pl.ds(base + hf * GW, GW), pl.ds(h * 128, 128)], wsem)

    for j in range(NCH):
        if j >= 2:
            wd[j - 2].wait()
        gd[j] = pltpu.async_copy(tab_hbm.at[j // NG].at[idx_v.at[j]],
                                 rows[j % 2], gsem)
        if j >= 1:
            gd[j - 1].wait()
            wd[j - 1] = _write(j - 1)
    gd[NCH - 1].wait()
    wd[NCH - 1] = _write(NCH - 1)
    wd[NCH - 2].wait()
    wd[NCH - 1].wait()
    for d in pd:
        d.wait()


def _sc_gather(ids_flat, tab_pair):
    mesh = plsc.VectorSubcoreMesh(core_axis_name="c", subcore_axis_name="s")
    f = pl.kernel(
        _sc_gather_body,
        mesh=mesh,
        out_type=(jax.ShapeDtypeStruct((BL, N_HEADS * 128), jnp.float32),
                  jax.ShapeDtypeStruct((N_HEADS, BL), jnp.float32)),
        scratch_types=[
            pltpu.VMEM((CHUNK + 16,), jnp.int32),
            pltpu.VMEM((NCH, GW), jnp.int32),
            pltpu.VMEM((NCH, GW), jnp.float32),
            pltpu.VMEM((GW, 128), jnp.float32),
            pltpu.VMEM((GW, 128), jnp.float32),
            pltpu.SemaphoreType.DMA,
            pltpu.SemaphoreType.DMA,
            pltpu.SemaphoreType.DMA,
        ],
        compiler_params=_sc_params(),
    )
    return f(ids_flat, tab_pair)


def _tc_body(ew_ref, par_ref, hid_ref, vwt_ref, kwt_ref, vb_ref, kb_ref,
             m_ref, cnw_ref, cw_ref, out_ref, tail_ref):
    i = pl.program_id(1)

    @pl.when(i == 0)
    def _():
        tail_ref[...] = jnp.zeros_like(tail_ref)

    ew = ew_ref[...]
    pmt = lax.dot_general(par_ref[...], jnp.eye(N_HEADS, dtype=jnp.float32),
                          (((0,), (0,)), ((), ())),
                          preferred_element_type=jnp.float32)
    pieces = []
    for h in range(N_HEADS):
        lo = ew[:, h * 128:h * 128 + 64]
        hi = ew[:, h * 128 + 64:h * 128 + 128]
        pm = pmt[:, h:h + 1]
        pieces.append(lo + pm * (hi - lo))
    emb = jnp.concatenate(pieces, axis=1)
    emb_bf = emb.astype(jnp.bfloat16)

    hid = hid_ref[...]
    val = jnp.dot(emb_bf, vwt_ref[...], preferred_element_type=jnp.float32)
    val = val + vb_ref[...]
    rq = lax.rsqrt(jnp.mean(hid * hid, axis=-1, keepdims=True) + 1e-6)
    kb = kb_ref[...]
    m = m_ref[...]
    cnw = cnw_ref[...]
    acc = jnp.zeros((T, D), jnp.float32)
    for h in range(HC):
        k = jnp.dot(emb_bf, kwt_ref[h], preferred_element_type=jnp.float32)
        k = k + kb[h:h + 1]
        rk = lax.rsqrt(jnp.mean(k * k, axis=-1, keepdims=True) + 1e-6)
        dkh = jnp.sum(k * hid * m[h:h + 1], axis=-1, keepdims=True)
        g = rk * rq * dkh * (1.0 / 32.0)
        gs = jnp.sqrt(jnp.maximum(jnp.abs(g), 1e-6)) * jnp.sign(g)
        gate = jax.nn.sigmoid(gs)
        vh = gate * val
        rv = lax.rsqrt(jnp.mean(vh * vh, axis=-1, keepdims=True) + 1e-5)
        xn = vh * rv * cnw[h:h + 1]
        ext = jnp.concatenate([tail_ref[h], xn], axis=0)
        cwh = cw_ref[h]
        y = ext[7:7 + T] * cwh[0:1]
        y = y + ext[10:10 + T] * cwh[1:2]
        y = y + ext[13:13 + T] * cwh[2:3]
        y = y + ext[16:16 + T] * cwh[3:4]
        conv = y * jax.nn.sigmoid(y)
        acc = acc + vh + conv
        tail_ref[h] = xn[T - 16:T]
    out_ref[...] = acc


def _tc_specs():
    def full(shape):
        return pl.BlockSpec(shape, lambda b, i, _n=len(shape): (0,) * _n)

    in_specs = [
        pl.BlockSpec((T, N_HEADS * 128), lambda b, i: (b * NL + i, 0)),
        pl.BlockSpec((N_HEADS, T), lambda b, i: (0, b * NL + i)),
        pl.BlockSpec((T, D), lambda b, i: (b * NL + i, 0)),
        full((E, D)),
        full((HC, E, D)),
        full((1, D)),
        full((HC, D)),
        full((HC, D)),
        full((HC, D)),
        full((HC, KCONV, D)),
    ]
    out_spec = pl.BlockSpec((T, D), lambda b, i: (b * NL + i, 0))
    return in_specs, out_spec


def _tc_call(ew, par, hid, vwt, kwt, vb, kb, m, cnw, cw):
    in_specs, out_spec = _tc_specs()
    return pl.pallas_call(
        _tc_body,
        grid=(B, NL),
        in_specs=in_specs,
        out_specs=out_spec,
        out_shape=jax.ShapeDtypeStruct((BL, D), jnp.float32),
        scratch_shapes=[pltpu.VMEM((HC, 16, D), jnp.float32)],
        compiler_params=pltpu.CompilerParams(
            dimension_semantics=("arbitrary", "arbitrary")),
    )(ew, par, hid, vwt, kwt, vb, kb, m, cnw, cw)


def kernel(hidden_states, tables, value_W, value_b, key_W, key_b,
           norm1_w, norm2_w, conv_w, conv_norm_w, input_ids):
    ids_flat = input_ids.reshape(BL)
    tabt = jnp.transpose(tables, (0, 2, 1))
    tab_pair = _sc_transpose(tabt)
    ew, par = _sc_gather(ids_flat, tab_pair)
    hid = hidden_states.reshape(BL, D)
    vwt = value_W.T.astype(jnp.bfloat16)
    kwt = jnp.transpose(key_W, (0, 2, 1)).astype(jnp.bfloat16)
    m = norm1_w * norm2_w
    cw = jnp.transpose(conv_w.reshape(HC, D, KCONV), (0, 2, 1))
    out = _tc_call(ew, par, hid, vwt, kwt, value_b.reshape(1, D), key_b,
                   m, conv_norm_w, cw)
    return out.reshape(B, L, D)


# speedup vs baseline: 1.4451x; 1.4451x over previous
import numpy as np
import jax
import jax.numpy as jnp
from jax import lax
from jax.experimental import pallas as pl
from jax.experimental.pallas import tpu as pltpu
from jax.experimental.pallas import tpu_sc as plsc

B, L, D = 4, 2048, 1024
VOCAB = 100000
N_HEADS = 16
HEAD_DIM = 64
E = N_HEADS * HEAD_DIM
HC = 4
KCONV = 4
PAD_ID = 2
BL = B * L

T = 256
NL = L // T
NW = 32
CHUNK = BL // NW
GW = 128
NG = CHUNK // GW
NCH = N_HEADS * NG


def _hash_coeffs() -> np.ndarray:
    rs = np.random.RandomState(0)
    m2 = rs.randint(1, 2 ** 31 - 1, size=(8, 2)).astype(np.uint64)
    m3 = rs.randint(1, 2 ** 31 - 1, size=(8, 3)).astype(np.uint64)
    mod = np.uint64(2 ** 32)
    p = np.uint64(1000003)
    c = np.zeros((16, 3), dtype=np.uint64)
    c[:8, 0] = (m2[:, 0] * p) % mod
    c[:8, 1] = m2[:, 1] % mod
    c[8:, 0] = (m3[:, 0] * p % mod * p) % mod
    c[8:, 1] = (m3[:, 1] * p) % mod
    c[8:, 2] = m3[:, 2] % mod
    return c.astype(np.uint32).view(np.int32).reshape(16, 3)


_C = _hash_coeffs()


def _sc_body(ids_hbm, tab3_hbm, out_hbm, ids_v, idx_v, rows0, rows1, gsem, wsem):
    wid = lax.axis_index("c") * 16 + lax.axis_index("s")
    base = wid * CHUNK

    pltpu.sync_copy(ids_hbm.at[pl.ds(base, CHUNK)], ids_v.at[pl.ds(16, CHUNK)])
    at_row_start = lax.rem(wid, L // CHUNK) == 0

    @pl.when(at_row_start)
    def _():
        ids_v[pl.ds(0, 16)] = jnp.full((16,), PAD_ID, jnp.int32)

    @pl.when(jnp.logical_not(at_row_start))
    def _():
        pltpu.sync_copy(ids_hbm.at[pl.ds(base - 16, 16)], ids_v.at[pl.ds(0, 16)])

    for h in range(N_HEADS):
        c0 = jnp.int32(int(_C[h, 0]))
        c1 = jnp.int32(int(_C[h, 1]))
        c2 = jnp.int32(int(_C[h, 2]))
        for hf in range(NG):

            @pl.loop(0, GW // 16)
            def _(i, h=h, hf=hf, c0=c0, c1=c1, c2=c2):
                s = 16 + hf * GW + i * 16
                t0 = ids_v[pl.ds(s, 16)]
                t1 = ids_v[pl.ds(s - 1, 16)]
                acc = t0 * c0 + t1 * c1
                if int(_C[h, 2]) != 0:
                    t2 = ids_v[pl.ds(s - 2, 16)]
                    acc = acc + t2 * c2
                accu = plsc.bitcast(acc, jnp.uint32)
                r = plsc.bitcast(accu % jnp.uint32(VOCAB), jnp.int32)
                idx_v[h * NG + hf, pl.ds(i * 16, 16)] = r

    rows = (rows0, rows1)
    gd = [None] * NCH
    wd = [None] * NCH

    def _write(j):
        h, hf = j // NG, j % NG
        return pltpu.async_copy(
            rows[j % 2],
            out_hbm.at[pl.ds(base + hf * GW, GW), pl.ds(h * HEAD_DIM, HEAD_DIM)],
            wsem)

    for j in range(NCH):
        if j >= 2:
            wd[j - 2].wait()
        gd[j] = pltpu.async_copy(tab3_hbm.at[j // NG].at[idx_v.at[j]],
                                 rows[j % 2], gsem)
        if j >= 1:
            gd[j - 1].wait()
            wd[j - 1] = _write(j - 1)
    gd[NCH - 1].wait()
    wd[NCH - 1] = _write(NCH - 1)
    wd[NCH - 2].wait()
    wd[NCH - 1].wait()


def _sc_gather(ids_flat, tab_flat):
    mesh = plsc.VectorSubcoreMesh(core_axis_name="c", subcore_axis_name="s")
    f = pl.kernel(
        _sc_body,
        mesh=mesh,
        out_type=jax.ShapeDtypeStruct((BL, E), jnp.float32),
        scratch_types=[
            pltpu.VMEM((CHUNK + 16,), jnp.int32),
            pltpu.VMEM((NCH, GW), jnp.int32),
            pltpu.VMEM((GW, HEAD_DIM), jnp.float32),
            pltpu.VMEM((GW, HEAD_DIM), jnp.float32),
            pltpu.SemaphoreType.DMA,
            pltpu.SemaphoreType.DMA,
        ],
        compiler_params=pltpu.CompilerParams(use_tc_tiling_on_sc=False),
    )
    return f(ids_flat, tab_flat)


def _tc_body(emb_ref, hid_ref, vwt_ref, kwt_ref, vb_ref, kb_ref, m_ref,
             cnw_ref, cw_ref, out_ref, tail_ref):
    i = pl.program_id(1)

    @pl.when(i == 0)
    def _():
        tail_ref[...] = jnp.zeros_like(tail_ref)

    emb_bf = emb_ref[...].astype(jnp.bfloat16)
    hid = hid_ref[...]
    val = jnp.dot(emb_bf, vwt_ref[...], preferred_element_type=jnp.float32)
    val = val + vb_ref[...]
    rq = lax.rsqrt(jnp.mean(hid * hid, axis=-1, keepdims=True) + 1e-6)
    kb = kb_ref[...]
    m = m_ref[...]
    cnw = cnw_ref[...]
    acc = jnp.zeros((T, D), jnp.float32)
    for h in range(HC):
        k = jnp.dot(emb_bf, kwt_ref[h], preferred_element_type=jnp.float32)
        k = k + kb[h:h + 1]
        rk = lax.rsqrt(jnp.mean(k * k, axis=-1, keepdims=True) + 1e-6)
        dkh = jnp.sum(k * hid * m[h:h + 1], axis=-1, keepdims=True)
        g = rk * rq * dkh * (1.0 / 32.0)
        gs = jnp.sqrt(jnp.maximum(jnp.abs(g), 1e-6)) * jnp.sign(g)
        gate = jax.nn.sigmoid(gs)
        vh = gate * val
        rv = lax.rsqrt(jnp.mean(vh * vh, axis=-1, keepdims=True) + 1e-5)
        xn = vh * rv * cnw[h:h + 1]
        ext = jnp.concatenate([tail_ref[h], xn], axis=0)
        cwh = cw_ref[h]
        y = ext[7:7 + T] * cwh[0:1]
        y = y + ext[10:10 + T] * cwh[1:2]
        y = y + ext[13:13 + T] * cwh[2:3]
        y = y + ext[16:16 + T] * cwh[3:4]
        conv = y * jax.nn.sigmoid(y)
        acc = acc + vh + conv
        tail_ref[h] = xn[T - 16:T]
    out_ref[...] = acc


def _tc_specs():
    def full(shape):
        return pl.BlockSpec(shape, lambda b, i, _n=len(shape): (0,) * _n)

    in_specs = [
        pl.BlockSpec((T, E), lambda b, i: (b * NL + i, 0)),
        pl.BlockSpec((T, D), lambda b, i: (b * NL + i, 0)),
        full((E, D)),
        full((HC, E, D)),
        full((1, D)),
        full((HC, D)),
        full((HC, D)),
        full((HC, D)),
        full((HC, KCONV, D)),
    ]
    out_spec = pl.BlockSpec((T, D), lambda b, i: (b * NL + i, 0))
    return in_specs, out_spec


def _tc_call(emb, hid, vwt, kwt, vb, kb, m, cnw, cw):
    in_specs, out_spec = _tc_specs()
    return pl.pallas_call(
        _tc_body,
        grid=(B, NL),
        in_specs=in_specs,
        out_specs=out_spec,
        out_shape=jax.ShapeDtypeStruct((BL, D), jnp.float32),
        scratch_shapes=[pltpu.VMEM((HC, 16, D), jnp.float32)],
        compiler_params=pltpu.CompilerParams(
            dimension_semantics=("arbitrary", "arbitrary")),
    )(emb, hid, vwt, kwt, vb, kb, m, cnw, cw)


def kernel(hidden_states, tables, value_W, value_b, key_W, key_b,
           norm1_w, norm2_w, conv_w, conv_norm_w, input_ids):
    ids_flat = input_ids.reshape(BL)
    emb = _sc_gather(ids_flat, tables)
    hid = hidden_states.reshape(BL, D)
    vwt = value_W.T.astype(jnp.bfloat16)
    kwt = jnp.transpose(key_W, (0, 2, 1)).astype(jnp.bfloat16)
    m = norm1_w * norm2_w
    cw = jnp.transpose(conv_w.reshape(HC, D, KCONV), (0, 2, 1))
    out = _tc_call(emb, hid, vwt, kwt, value_b.reshape(1, D), key_b,
                   m, conv_norm_w, cw)
    return out.reshape(B, L, D)


# speedup vs baseline: 1.4585x; 1.0092x over previous
import numpy as np
import jax
import jax.numpy as jnp
from jax import lax
from jax.experimental import pallas as pl
from jax.experimental.pallas import tpu as pltpu
from jax.experimental.pallas import tpu_sc as plsc

B, L, D = 4, 2048, 1024
VOCAB = 100000
N_HEADS = 16
HEAD_DIM = 64
E = N_HEADS * HEAD_DIM
HC = 4
KCONV = 4
PAD_ID = 2
BL = B * L

T = 512
NL = L // T
NW = 32
CHUNK = BL // NW
GW = 128
NG = CHUNK // GW
NCH = N_HEADS * NG


def _hash_coeffs() -> np.ndarray:
    rs = np.random.RandomState(0)
    m2 = rs.randint(1, 2 ** 31 - 1, size=(8, 2)).astype(np.uint64)
    m3 = rs.randint(1, 2 ** 31 - 1, size=(8, 3)).astype(np.uint64)
    mod = np.uint64(2 ** 32)
    p = np.uint64(1000003)
    c = np.zeros((16, 3), dtype=np.uint64)
    c[:8, 0] = (m2[:, 0] * p) % mod
    c[:8, 1] = m2[:, 1] % mod
    c[8:, 0] = (m3[:, 0] * p % mod * p) % mod
    c[8:, 1] = (m3[:, 1] * p) % mod
    c[8:, 2] = m3[:, 2] % mod
    return c.astype(np.uint32).view(np.int32).reshape(16, 3)


_C = _hash_coeffs()


def _sc_body(ids_hbm, tab3_hbm, out_hbm, ids_v, idx_v, rows0, rows1, gsem, wsem):
    wid = lax.axis_index("c") * 16 + lax.axis_index("s")
    base = wid * CHUNK

    pltpu.sync_copy(ids_hbm.at[pl.ds(base, CHUNK)], ids_v.at[pl.ds(16, CHUNK)])
    at_row_start = lax.rem(wid, L // CHUNK) == 0

    @pl.when(at_row_start)
    def _():
        ids_v[pl.ds(0, 16)] = jnp.full((16,), PAD_ID, jnp.int32)

    @pl.when(jnp.logical_not(at_row_start))
    def _():
        pltpu.sync_copy(ids_hbm.at[pl.ds(base - 16, 16)], ids_v.at[pl.ds(0, 16)])

    for h in range(N_HEADS):
        c0 = jnp.int32(int(_C[h, 0]))
        c1 = jnp.int32(int(_C[h, 1]))
        c2 = jnp.int32(int(_C[h, 2]))
        for hf in range(NG):

            @pl.loop(0, GW // 16)
            def _(i, h=h, hf=hf, c0=c0, c1=c1, c2=c2):
                s = 16 + hf * GW + i * 16
                t0 = ids_v[pl.ds(s, 16)]
                t1 = ids_v[pl.ds(s - 1, 16)]
                acc = t0 * c0 + t1 * c1
                if int(_C[h, 2]) != 0:
                    t2 = ids_v[pl.ds(s - 2, 16)]
                    acc = acc + t2 * c2
                accu = plsc.bitcast(acc, jnp.uint32)
                r = plsc.bitcast(accu % jnp.uint32(VOCAB), jnp.int32)
                idx_v[h * NG + hf, pl.ds(i * 16, 16)] = r

    rows = (rows0, rows1)
    gd = [None] * NCH
    wd = [None] * NCH

    def _write(j):
        h, hf = j // NG, j % NG
        return pltpu.async_copy(
            rows[j % 2],
            out_hbm.at[pl.ds(base + hf * GW, GW), pl.ds(h * HEAD_DIM, HEAD_DIM)],
            wsem)

    for j in range(NCH):
        if j >= 2:
            wd[j - 2].wait()
        gd[j] = pltpu.async_copy(tab3_hbm.at[j // NG].at[idx_v.at[j]],
                                 rows[j % 2], gsem)
        if j >= 1:
            gd[j - 1].wait()
            wd[j - 1] = _write(j - 1)
    gd[NCH - 1].wait()
    wd[NCH - 1] = _write(NCH - 1)
    wd[NCH - 2].wait()
    wd[NCH - 1].wait()


def _sc_gather(ids_flat, tab_flat):
    mesh = plsc.VectorSubcoreMesh(core_axis_name="c", subcore_axis_name="s")
    f = pl.kernel(
        _sc_body,
        mesh=mesh,
        out_type=jax.ShapeDtypeStruct((BL, E), jnp.float32),
        scratch_types=[
            pltpu.VMEM((CHUNK + 16,), jnp.int32),
            pltpu.VMEM((NCH, GW), jnp.int32),
            pltpu.VMEM((GW, HEAD_DIM), jnp.float32),
            pltpu.VMEM((GW, HEAD_DIM), jnp.float32),
            pltpu.SemaphoreType.DMA,
            pltpu.SemaphoreType.DMA,
        ],
        compiler_params=pltpu.CompilerParams(use_tc_tiling_on_sc=False),
    )
    return f(ids_flat, tab_flat)


def _tc_body(emb_ref, hid_ref, vwt_ref, kwt_ref, vb_ref, kb_ref, m_ref,
             cnw_ref, cw_ref, out_ref, tail_ref):
    i = pl.program_id(1)

    @pl.when(i == 0)
    def _():
        tail_ref[...] = jnp.zeros_like(tail_ref)

    emb_bf = emb_ref[...].astype(jnp.bfloat16)
    hid = hid_ref[...]
    val = jnp.dot(emb_bf, vwt_ref[...], preferred_element_type=jnp.float32)
    val = val + vb_ref[...]
    rq = lax.rsqrt(jnp.mean(hid * hid, axis=-1, keepdims=True) + 1e-6)
    kb = kb_ref[...]
    m = m_ref[...]
    cnw = cnw_ref[...]
    acc = jnp.zeros((T, D), jnp.float32)
    for h in range(HC):
        k = jnp.dot(emb_bf, kwt_ref[h], preferred_element_type=jnp.float32)
        k = k + kb[h:h + 1]
        rk = lax.rsqrt(jnp.mean(k * k, axis=-1, keepdims=True) + 1e-6)
        dkh = jnp.sum(k * hid * m[h:h + 1], axis=-1, keepdims=True)
        g = rk * rq * dkh * (1.0 / 32.0)
        gs = jnp.sqrt(jnp.maximum(jnp.abs(g), 1e-6)) * jnp.sign(g)
        gate = jax.nn.sigmoid(gs)
        vh = gate * val
        rv = lax.rsqrt(jnp.mean(vh * vh, axis=-1, keepdims=True) + 1e-5)
        xn = vh * rv * cnw[h:h + 1]
        ext = jnp.concatenate([tail_ref[h], xn], axis=0)
        cwh = cw_ref[h]
        y = ext[7:7 + T] * cwh[0:1]
        y = y + ext[10:10 + T] * cwh[1:2]
        y = y + ext[13:13 + T] * cwh[2:3]
        y = y + ext[16:16 + T] * cwh[3:4]
        conv = y * jax.nn.sigmoid(y)
        acc = acc + vh + conv
        tail_ref[h] = xn[T - 16:T]
    out_ref[...] = acc


def _tc_specs():
    def full(shape):
        return pl.BlockSpec(shape, lambda b, i, _n=len(shape): (0,) * _n)

    in_specs = [
        pl.BlockSpec((T, E), lambda b, i: (b * NL + i, 0)),
        pl.BlockSpec((T, D), lambda b, i: (b * NL + i, 0)),
        full((E, D)),
        full((HC, E, D)),
        full((1, D)),
        full((HC, D)),
        full((HC, D)),
        full((HC, D)),
        full((HC, KCONV, D)),
    ]
    out_spec = pl.BlockSpec((T, D), lambda b, i: (b * NL + i, 0))
    return in_specs, out_spec


def _tc_call(emb, hid, vwt, kwt, vb, kb, m, cnw, cw):
    in_specs, out_spec = _tc_specs()
    return pl.pallas_call(
        _tc_body,
        grid=(B, NL),
        in_specs=in_specs,
        out_specs=out_spec,
        out_shape=jax.ShapeDtypeStruct((BL, D), jnp.float32),
        scratch_shapes=[pltpu.VMEM((HC, 16, D), jnp.float32)],
        compiler_params=pltpu.CompilerParams(
            dimension_semantics=("arbitrary", "arbitrary")),
    )(emb, hid, vwt, kwt, vb, kb, m, cnw, cw)


def kernel(hidden_states, tables, value_W, value_b, key_W, key_b,
           norm1_w, norm2_w, conv_w, conv_norm_w, input_ids):
    ids_flat = input_ids.reshape(BL)
    emb = _sc_gather(ids_flat, tables)
    hid = hidden_states.reshape(BL, D)
    vwt = value_W.T.astype(jnp.bfloat16)
    kwt = jnp.transpose(key_W, (0, 2, 1)).astype(jnp.bfloat16)
    m = norm1_w * norm2_w
    cw = jnp.transpose(conv_w.reshape(HC, D, KCONV), (0, 2, 1))
    out = _tc_call(emb, hid, vwt, kwt, value_b.reshape(1, D), key_b,
                   m, conv_norm_w, cw)
    return out.reshape(B, L, D)
